# Initial kernel scaffold; baseline (speedup 1.0000x reference)
import jax
import jax.numpy as jnp
from jax.experimental import pallas as pl


def kernel(input_ids, weight):
    raise NotImplementedError("write your pallas kernel here")


# speedup vs baseline: 1.6836x; 1.6836x over previous
import functools

import jax
import jax.numpy as jnp
from jax import lax
from jax.experimental import pallas as pl
from jax.experimental.pallas import tpu as pltpu
from jax.experimental.pallas import tpu_sc as plsc

D = 64
NC = 2
NS = 16
NW = NC * NS
C = 128
B = 16384 * 50
BPW = B // NW
NCHUNK = BPW // C


@functools.partial(
    pl.kernel,
    out_type=jax.ShapeDtypeStruct((B, D), jnp.float32),
    mesh=plsc.VectorSubcoreMesh(core_axis_name="c", subcore_axis_name="s"),
    compiler_params=pltpu.CompilerParams(use_tc_tiling_on_sc=False),
    scratch_types=[
        pltpu.VMEM((NCHUNK, C), jnp.int32),
        pltpu.VMEM((C, D), jnp.float32),
        pltpu.SemaphoreType.DMA,
    ],
)
def _emb_lookup(idx_hbm, table_hbm, out_hbm, idx_v, rows_v, gsem):
    wid = lax.axis_index("s") * NC + lax.axis_index("c")
    base = wid * BPW
    pltpu.sync_copy(idx_hbm.at[wid], idx_v)

    def body(g, carry):
        pltpu.async_copy(table_hbm.at[idx_v.at[g]], rows_v, gsem).wait()
        pltpu.sync_copy(rows_v, out_hbm.at[pl.ds(base + g * C, C)])
        return carry

    lax.fori_loop(0, NCHUNK, body, 0)


def kernel(input_ids, weight):
    bsz, hist = input_ids.shape
    idx = input_ids.reshape(NW, NCHUNK, C).astype(jnp.int32)
    out = _emb_lookup(idx, weight)
    return out.reshape(bsz, hist, D)


# speedup vs baseline: 1.8781x; 1.1155x over previous
import functools

import jax
import jax.numpy as jnp
from jax import lax
from jax.experimental import pallas as pl
from jax.experimental.pallas import tpu as pltpu
from jax.experimental.pallas import tpu_sc as plsc

D = 64
NC = 2
NS = 16
NW = NC * NS
C = 128
B = 16384 * 50
BPW = B // NW
NCHUNK = BPW // C
NBUF = 4
NGRP = NCHUNK // NBUF


@functools.partial(
    pl.kernel,
    out_type=jax.ShapeDtypeStruct((B, D), jnp.float32),
    mesh=plsc.VectorSubcoreMesh(core_axis_name="c", subcore_axis_name="s"),
    compiler_params=pltpu.CompilerParams(use_tc_tiling_on_sc=False),
    scratch_types=(
        [pltpu.VMEM((NCHUNK, C), jnp.int32)]
        + [pltpu.VMEM((C, D), jnp.float32) for _ in range(NBUF)]
        + [pltpu.SemaphoreType.DMA for _ in range(2 * NBUF)]
    ),
)
def _emb_lookup(idx_hbm, table_hbm, out_hbm, idx_v, *bufs):
    rows = bufs[:NBUF]
    gsem = bufs[NBUF:2 * NBUF]
    wsem = bufs[2 * NBUF:]
    wid = lax.axis_index("s") * NC + lax.axis_index("c")
    base = wid * BPW
    pltpu.sync_copy(idx_hbm.at[wid], idx_v)

    for b in range(NBUF):
        pltpu.async_copy(table_hbm.at[idx_v.at[b]], rows[b], gsem[b])

    def grp_body(grp, carry):
        g0 = grp * NBUF
        for b in range(NBUF):
            g = g0 + b
            dst = out_hbm.at[pl.ds(base + g * C, C)]
            pltpu.make_async_copy(table_hbm.at[idx_v.at[g]], rows[b], gsem[b]).wait()
            pltpu.async_copy(rows[b], dst, wsem[b])

            @pl.when(grp < NGRP - 1)
            def _():
                pltpu.make_async_copy(rows[b], dst, wsem[b]).wait()
                pltpu.async_copy(table_hbm.at[idx_v.at[g + NBUF]], rows[b], gsem[b])

        return carry

    lax.fori_loop(0, NGRP, grp_body, 0)

    for b in range(NBUF):
        g = NCHUNK - NBUF + b
        dst = out_hbm.at[pl.ds(base + g * C, C)]
        pltpu.make_async_copy(rows[b], dst, wsem[b]).wait()


def kernel(input_ids, weight):
    bsz, hist = input_ids.shape
    idx = input_ids.reshape(NW, NCHUNK, C).astype(jnp.int32)
    out = _emb_lookup(idx, weight)
    return out.reshape(bsz, hist, D)
